# Initial kernel scaffold; baseline (speedup 1.0000x reference)
import jax
import jax.numpy as jnp
from jax.experimental import pallas as pl


def kernel(x, edge_index, W1, b1, W2, b2):
    raise NotImplementedError("write your pallas kernel here")


# speedup vs baseline: 21.7667x; 21.7667x over previous
import functools

import jax
import jax.numpy as jnp
from jax import lax
from jax.experimental import pallas as pl
from jax.experimental.pallas import tpu as pltpu
from jax.experimental.pallas import tpu_sc as plsc

N = 10000
E = 320000
IN_CH = 128
HID_CH = 128
OUT_CH = 64

NC, NS, L = 2, 16, 16
CHUNK = 128
NPAD = 10240
RPT = NPAD // NS
NCHUNK = 158
EPW = NCHUNK * CHUNK
EPAD = EPW * NS
DEG_C = 8


@functools.cache
def _build_sc_degree():
  HALF = NCHUNK // 2

  @functools.partial(
      pl.kernel,
      out_type=jax.ShapeDtypeStruct((NC, NPAD, DEG_C), jnp.float32),
      mesh=plsc.VectorSubcoreMesh(core_axis_name="c", subcore_axis_name="s"),
      compiler_params=pltpu.CompilerParams(use_tc_tiling_on_sc=False),
      scratch_types=[
          pltpu.VMEM((HALF, CHUNK), jnp.int32),
          pltpu.VMEM((CHUNK, DEG_C), jnp.float32),
          pltpu.VMEM_SHARED((NPAD, DEG_C), jnp.float32),
          pltpu.SemaphoreType.DMA,
          pltpu.SemaphoreType.DMA,
      ],
  )
  def _sc_degree(dst_hbm, const_hbm, out_hbm, didx, buf, acc, sem0, sem1):
      c = lax.axis_index("c")
      s = lax.axis_index("s")

      pltpu.sync_copy(dst_hbm.at[s, pl.ds(c * HALF, HALF)], didx)

      pltpu.sync_copy(const_hbm.at[0], buf)
      def zrow(j, _):
          pltpu.sync_copy(buf, acc.at[pl.ds(s * RPT + j * CHUNK, CHUNK)])
          return 0
      lax.fori_loop(0, RPT // CHUNK, zrow, 0)
      plsc.subcore_barrier()

      pltpu.sync_copy(const_hbm.at[1], buf)
      def step(t, _):
          d0 = pltpu.async_copy(buf, acc.at[didx.at[2 * t]], sem0, add=True)
          d1 = pltpu.async_copy(buf, acc.at[didx.at[2 * t + 1]], sem1, add=True)
          d0.wait()
          d1.wait()
          return 0
      lax.fori_loop(0, HALF // 2, step, 0)
      pltpu.sync_copy(buf, acc.at[didx.at[HALF - 1]], add=True)
      plsc.subcore_barrier()

      pltpu.sync_copy(acc.at[pl.ds(s * RPT, RPT)], out_hbm.at[c, pl.ds(s * RPT, RPT)])

  return _sc_degree


@functools.cache
def _build_sc_aggregate(C2):
    @functools.partial(
        pl.kernel,
        out_type=jax.ShapeDtypeStruct((NC, NPAD, C2), jnp.float32),
        mesh=plsc.VectorSubcoreMesh(core_axis_name="c", subcore_axis_name="s"),
        compiler_params=pltpu.CompilerParams(use_tc_tiling_on_sc=False),
        scratch_types=[
            pltpu.VMEM((NCHUNK, CHUNK), jnp.int32),
            pltpu.VMEM((NCHUNK, CHUNK), jnp.int32),
            pltpu.VMEM((CHUNK, C2), jnp.float32),
            pltpu.VMEM((CHUNK, C2), jnp.float32),
            pltpu.VMEM_SHARED((NPAD, C2), jnp.float32),
            pltpu.SemaphoreType.DMA,
            pltpu.SemaphoreType.DMA,
        ],
    )
    def agg(h_hbm, src_hbm, dst_hbm, zeros_hbm, out_hbm, sidx, didx,
            rows0, rows1, acc, sem0, sem1):
        c = lax.axis_index("c")
        s = lax.axis_index("s")
        h_half = h_hbm.at[c]

        pltpu.sync_copy(src_hbm.at[s], sidx)
        pltpu.sync_copy(dst_hbm.at[s], didx)

        pltpu.sync_copy(zeros_hbm, rows0)
        def zacc(j, _):
            pltpu.sync_copy(rows0, acc.at[pl.ds(s * RPT + j * CHUNK, CHUNK)])
            return 0
        lax.fori_loop(0, RPT // CHUNK, zacc, 0)
        plsc.subcore_barrier()

        pltpu.async_copy(h_half.at[sidx.at[0]], rows0, sem0)

        def step(t, _):
            k0 = 2 * t
            pltpu.async_copy(h_half.at[sidx.at[k0 + 1]], rows1, sem1)
            pltpu.make_async_copy(h_half.at[sidx.at[k0]], rows0, sem0).wait()
            pltpu.sync_copy(rows0, acc.at[didx.at[k0]], add=True)
            pltpu.async_copy(h_half.at[sidx.at[(k0 + 2) % NCHUNK]], rows0, sem0)
            pltpu.make_async_copy(h_half.at[sidx.at[k0 + 1]], rows1, sem1).wait()
            pltpu.sync_copy(rows1, acc.at[didx.at[k0 + 1]], add=True)
            return 0
        lax.fori_loop(0, NCHUNK // 2, step, 0)
        pltpu.make_async_copy(h_half.at[sidx.at[0]], rows0, sem0).wait()
        plsc.subcore_barrier()

        pltpu.sync_copy(acc.at[pl.ds(s * RPT, RPT)], out_hbm.at[c, pl.ds(s * RPT, RPT)])

    return agg


BM = 1000


def _tc1_body(x_ref, w_ref, degp_ref, hp_ref, q_ref, dinv_ref):
    deg = degp_ref[0, :, 0] + degp_ref[1, :, 0] + 1.0
    dinv = lax.rsqrt(deg)[:, None]
    h = jnp.dot(x_ref[...], w_ref[...], preferred_element_type=jnp.float32)
    hp = h * dinv
    hp_ref[0] = hp[:, : HID_CH // 2]
    hp_ref[1] = hp[:, HID_CH // 2 :]
    q_ref[...] = h * (dinv * dinv)
    dinv_ref[...] = jnp.broadcast_to(dinv, (BM, 8))


def _tc1(x, W1, degp):
    return pl.pallas_call(
        _tc1_body,
        grid=(N // BM,),
        in_specs=[
            pl.BlockSpec((BM, IN_CH), lambda i: (i, 0)),
            pl.BlockSpec((IN_CH, HID_CH), lambda i: (0, 0)),
            pl.BlockSpec((NC, BM, DEG_C), lambda i: (0, i, 0)),
        ],
        out_specs=[
            pl.BlockSpec((NC, BM, HID_CH // 2), lambda i: (0, i, 0)),
            pl.BlockSpec((BM, HID_CH), lambda i: (i, 0)),
            pl.BlockSpec((BM, 8), lambda i: (i, 0)),
        ],
        out_shape=[
            jax.ShapeDtypeStruct((NC, N, HID_CH // 2), jnp.float32),
            jax.ShapeDtypeStruct((N, HID_CH), jnp.float32),
            jax.ShapeDtypeStruct((N, 8), jnp.float32),
        ],
    )(x, W1, degp)


def _tc2_body(agg_ref, dinv_ref, q_ref, b_ref, w_ref, hp_ref, q2_ref):
    dinv = dinv_ref[:, 0:1]
    aggsum = jnp.concatenate([agg_ref[0], agg_ref[1]], axis=1)
    z = jnp.maximum(aggsum * dinv + q_ref[...] + b_ref[...], 0.0)
    h2 = jnp.dot(z, w_ref[...], preferred_element_type=jnp.float32)
    hp = h2 * dinv
    hp_ref[0] = hp[:, : OUT_CH // 2]
    hp_ref[1] = hp[:, OUT_CH // 2 :]
    q2_ref[...] = h2 * (dinv * dinv)


def _tc2(agg1, dinv8, q1, b1, W2):
    return pl.pallas_call(
        _tc2_body,
        grid=(N // BM,),
        in_specs=[
            pl.BlockSpec((NC, BM, HID_CH // 2), lambda i: (0, i, 0)),
            pl.BlockSpec((BM, 8), lambda i: (i, 0)),
            pl.BlockSpec((BM, HID_CH), lambda i: (i, 0)),
            pl.BlockSpec((1, HID_CH), lambda i: (0, 0)),
            pl.BlockSpec((HID_CH, OUT_CH), lambda i: (0, 0)),
        ],
        out_specs=[
            pl.BlockSpec((NC, BM, OUT_CH // 2), lambda i: (0, i, 0)),
            pl.BlockSpec((BM, OUT_CH), lambda i: (i, 0)),
        ],
        out_shape=[
            jax.ShapeDtypeStruct((NC, N, OUT_CH // 2), jnp.float32),
            jax.ShapeDtypeStruct((N, OUT_CH), jnp.float32),
        ],
    )(agg1, dinv8, q1, b1, W2)


def _tc3_body(agg_ref, dinv_ref, q_ref, b_ref, out_ref):
    dinv = dinv_ref[:, 0:1]
    aggsum = jnp.concatenate([agg_ref[0], agg_ref[1]], axis=1)
    out_ref[...] = aggsum * dinv + q_ref[...] + b_ref[...]


def _tc3(agg2, dinv8, q2, b2):
    return pl.pallas_call(
        _tc3_body,
        grid=(N // BM,),
        in_specs=[
            pl.BlockSpec((NC, BM, OUT_CH // 2), lambda i: (0, i, 0)),
            pl.BlockSpec((BM, 8), lambda i: (i, 0)),
            pl.BlockSpec((BM, OUT_CH), lambda i: (i, 0)),
            pl.BlockSpec((1, OUT_CH), lambda i: (0, 0)),
        ],
        out_specs=pl.BlockSpec((BM, OUT_CH), lambda i: (i, 0)),
        out_shape=jax.ShapeDtypeStruct((N, OUT_CH), jnp.float32),
    )(agg2, dinv8, q2, b2)


def kernel(x, edge_index, W1, b1, W2, b2):
    src = edge_index[0].astype(jnp.int32)
    dst = edge_index[1].astype(jnp.int32)
    pad = EPAD - E
    srcp = jnp.concatenate([src, jnp.zeros((pad,), jnp.int32)])
    srcp = srcp.reshape(NS, NCHUNK, CHUNK)
    dstp = jnp.concatenate([dst, jnp.full((pad,), N, jnp.int32)])
    dstp = dstp.reshape(NS, NCHUNK, CHUNK)

    deg_const = jnp.stack([jnp.zeros((CHUNK, DEG_C), jnp.float32),
                           jnp.ones((CHUNK, DEG_C), jnp.float32)])
    zeros_hid = jnp.zeros((CHUNK, HID_CH // 2), jnp.float32)
    zeros_out = jnp.zeros((CHUNK, OUT_CH // 2), jnp.float32)

    degp = _build_sc_degree()(dstp, deg_const)
    h1p, q1, dinv8 = _tc1(x, W1, degp)
    agg1 = _build_sc_aggregate(HID_CH // 2)(h1p, srcp, dstp, zeros_hid)
    h2p, q2 = _tc2(agg1, dinv8, q1, b1.reshape(1, HID_CH), W2)
    agg2 = _build_sc_aggregate(OUT_CH // 2)(h2p, srcp, dstp, zeros_out)
    return _tc3(agg2, dinv8, q2, b2.reshape(1, OUT_CH))
